# Initial kernel scaffold; baseline (speedup 1.0000x reference)
import jax
import jax.numpy as jnp
from jax.experimental import pallas as pl


def kernel(x, W_enc, b_enc, W_dec, b_dec):
    raise NotImplementedError("write your pallas kernel here")


# speedup vs baseline: 3.8601x; 3.8601x over previous
import functools

import jax
import jax.numpy as jnp
from jax.experimental import pallas as pl
from jax.experimental.pallas import tpu as pltpu

INPUT_DIM = 2048
DICT_SIZE = 32768
K = 64
N_TOKENS = 128

_ENC_BD = 2048
_DEC_BD = 2048
_TOPK_BR = 16


def _enc_body(x_ref, w_ref, b_ref, out_ref):
    out_ref[...] = jax.lax.dot_general(
        x_ref[...], w_ref[...],
        (((1,), (1,)), ((), ())),
        preferred_element_type=jnp.float32,
    ) + b_ref[...][None, :]


def _topk_body(pa_ref, acts_ref):
    v = pa_ref[...]
    bits = jax.lax.bitcast_convert_type(v, jnp.uint32)
    u = jnp.where(v >= 0.0, bits | jnp.uint32(0x80000000), ~bits)
    thr = jnp.zeros((v.shape[0], 1), jnp.uint32)
    for b in range(31, -1, -1):
        cand = thr | jnp.uint32(1 << b)
        cnt = jnp.sum((u >= cand).astype(jnp.int32), axis=1, keepdims=True)
        thr = jnp.where(cnt >= K, cand, thr)
    mask = u >= thr
    acts_ref[...] = jnp.where(mask, jnp.maximum(v, 0.0), 0.0)


def _dec_body(acts_ref, w_ref, b_ref, out_ref):
    @pl.when(pl.program_id(0) == 0)
    def _init():
        out_ref[...] = jnp.broadcast_to(b_ref[...][None, :], out_ref.shape)

    out_ref[...] += jax.lax.dot_general(
        acts_ref[...], w_ref[...],
        (((1,), (0,)), ((), ())),
        preferred_element_type=jnp.float32,
    )


def kernel(x, W_enc, b_enc, W_dec, b_dec):
    x_cent = x - b_dec[None, :]

    pre_acts = pl.pallas_call(
        _enc_body,
        grid=(DICT_SIZE // _ENC_BD,),
        in_specs=[
            pl.BlockSpec((N_TOKENS, INPUT_DIM), lambda d: (0, 0)),
            pl.BlockSpec((_ENC_BD, INPUT_DIM), lambda d: (d, 0)),
            pl.BlockSpec((_ENC_BD,), lambda d: (d,)),
        ],
        out_specs=pl.BlockSpec((N_TOKENS, _ENC_BD), lambda d: (0, d)),
        out_shape=jax.ShapeDtypeStruct((N_TOKENS, DICT_SIZE), jnp.float32),
    )(x_cent, W_enc, b_enc)

    acts = pl.pallas_call(
        _topk_body,
        grid=(N_TOKENS // _TOPK_BR,),
        in_specs=[pl.BlockSpec((_TOPK_BR, DICT_SIZE), lambda r: (r, 0))],
        out_specs=pl.BlockSpec((_TOPK_BR, DICT_SIZE), lambda r: (r, 0)),
        out_shape=jax.ShapeDtypeStruct((N_TOKENS, DICT_SIZE), jnp.float32),
    )(pre_acts)

    recon = pl.pallas_call(
        _dec_body,
        grid=(DICT_SIZE // _DEC_BD,),
        in_specs=[
            pl.BlockSpec((N_TOKENS, _DEC_BD), lambda d: (0, d)),
            pl.BlockSpec((_DEC_BD, INPUT_DIM), lambda d: (d, 0)),
            pl.BlockSpec((INPUT_DIM,), lambda d: (0,)),
        ],
        out_specs=pl.BlockSpec((N_TOKENS, INPUT_DIM), lambda d: (0, 0)),
        out_shape=jax.ShapeDtypeStruct((N_TOKENS, INPUT_DIM), jnp.float32),
    )(acts, W_dec, b_dec)

    return (recon, acts)
